# Initial kernel scaffold; baseline (speedup 1.0000x reference)
import jax
import jax.numpy as jnp
from jax.experimental import pallas as pl


def kernel(x, edge_index, W_self0, W_neigh0, b0, W_self1, W_neigh1, b1, W_self2, W_neigh2, b2):
    raise NotImplementedError("write your pallas kernel here")


# speedup vs baseline: 2.7514x; 2.7514x over previous
import jax
import jax.numpy as jnp
from jax import lax
from jax.experimental import pallas as pl
from jax.experimental.pallas import tpu as pltpu
from jax.experimental.pallas import tpu_sc as plsc

N = 10000
E = 160000
D_IN = 256
D_H = 256
D_OUT = 40
D_OUT_PAD = 128

NC = 2
NS = 16
CK = 64
CHUNKS = 160
E_PAD = NS * CHUNKS * CK
N_PAD = 10240
ROWS_PER_TILE = N_PAD // NS
PAD_DST = 10016
RB = 512
GRID = N_PAD // RB


def _tc_mm0(x_ref, ws_ref, wn_ref, b_ref, s_ref, m_ref):
    xb = x_ref[...]
    s_ref[...] = jnp.dot(xb, ws_ref[...], preferred_element_type=jnp.float32) + b_ref[...]
    mm = jnp.dot(xb, wn_ref[...], preferred_element_type=jnp.float32)
    half = mm.shape[1] // 2
    m_ref[0] = mm[:, :half]
    m_ref[1] = mm[:, half:]


def _tc_comb(s_ref, a_ref, d_ref, ws_ref, wn_ref, b_ref, s_out, m_out):
    deg = jnp.maximum(d_ref[0, :, 0:1] + d_ref[1, :, 0:1], 1.0)
    h = s_ref[...] + jnp.concatenate([a_ref[0], a_ref[1]], axis=1) / deg
    h = jnp.maximum(h, 0.0)
    s_out[...] = jnp.dot(h, ws_ref[...], preferred_element_type=jnp.float32) + b_ref[...]
    mm = jnp.dot(h, wn_ref[...], preferred_element_type=jnp.float32)
    half = mm.shape[1] // 2
    m_out[0] = mm[:, :half]
    m_out[1] = mm[:, half:]


def _tc_comb_flat(s_ref, a_ref, d_ref, ws_ref, wn_ref, b_ref, s_out, m_out):
    deg = jnp.maximum(d_ref[0, :, 0:1] + d_ref[1, :, 0:1], 1.0)
    h = s_ref[...] + jnp.concatenate([a_ref[0], a_ref[1]], axis=1) / deg
    h = jnp.maximum(h, 0.0)
    s_out[...] = jnp.dot(h, ws_ref[...], preferred_element_type=jnp.float32) + b_ref[...]
    m_out[...] = jnp.dot(h, wn_ref[...], preferred_element_type=jnp.float32)


def _tc_fin(s_ref, a_ref, d_ref, o_ref):
    deg = jnp.maximum(d_ref[0, :, 0:1] + d_ref[1, :, 0:1], 1.0)
    o_ref[...] = s_ref[...] + (a_ref[0] + a_ref[1]) / deg


def _mm0_call(x, ws, wn, b):
    return pl.pallas_call(
        _tc_mm0,
        grid=(GRID,),
        in_specs=[
            pl.BlockSpec((RB, D_IN), lambda i: (i, 0)),
            pl.BlockSpec((D_IN, D_H), lambda i: (0, 0)),
            pl.BlockSpec((D_IN, D_H), lambda i: (0, 0)),
            pl.BlockSpec((1, D_H), lambda i: (0, 0)),
        ],
        out_specs=[
            pl.BlockSpec((RB, D_H), lambda i: (i, 0)),
            pl.BlockSpec((2, RB, D_H // 2), lambda i: (0, i, 0)),
        ],
        out_shape=[
            jax.ShapeDtypeStruct((N_PAD, D_H), jnp.float32),
            jax.ShapeDtypeStruct((2, N_PAD, D_H // 2), jnp.float32),
        ],
    )(x, ws, wn, b)


def _comb_call(s, a, d, ws, wn, b, d_out):
    d_in = s.shape[1]
    return pl.pallas_call(
        _tc_comb,
        grid=(GRID,),
        in_specs=[
            pl.BlockSpec((RB, d_in), lambda i: (i, 0)),
            pl.BlockSpec((2, RB, d_in // 2), lambda i: (0, i, 0)),
            pl.BlockSpec((2, RB, 128), lambda i: (0, i, 0)),
            pl.BlockSpec((d_in, d_out), lambda i: (0, 0)),
            pl.BlockSpec((d_in, d_out), lambda i: (0, 0)),
            pl.BlockSpec((1, d_out), lambda i: (0, 0)),
        ],
        out_specs=[
            pl.BlockSpec((RB, d_out), lambda i: (i, 0)),
            pl.BlockSpec((2, RB, d_out // 2), lambda i: (0, i, 0)),
        ],
        out_shape=[
            jax.ShapeDtypeStruct((N_PAD, d_out), jnp.float32),
            jax.ShapeDtypeStruct((2, N_PAD, d_out // 2), jnp.float32),
        ],
    )(s, a, d, ws, wn, b)


def _comb_flat_call(s, a, d, ws, wn, b, d_out):
    d_in = s.shape[1]
    return pl.pallas_call(
        _tc_comb_flat,
        grid=(GRID,),
        in_specs=[
            pl.BlockSpec((RB, d_in), lambda i: (i, 0)),
            pl.BlockSpec((2, RB, d_in // 2), lambda i: (0, i, 0)),
            pl.BlockSpec((2, RB, 128), lambda i: (0, i, 0)),
            pl.BlockSpec((d_in, d_out), lambda i: (0, 0)),
            pl.BlockSpec((d_in, d_out), lambda i: (0, 0)),
            pl.BlockSpec((1, d_out), lambda i: (0, 0)),
        ],
        out_specs=[
            pl.BlockSpec((RB, d_out), lambda i: (i, 0)),
            pl.BlockSpec((RB, d_out), lambda i: (i, 0)),
        ],
        out_shape=[
            jax.ShapeDtypeStruct((N_PAD, d_out), jnp.float32),
            jax.ShapeDtypeStruct((N_PAD, d_out), jnp.float32),
        ],
    )(s, a, d, ws, wn, b)


def _fin_call(s, a, d):
    d_in = s.shape[1]
    return pl.pallas_call(
        _tc_fin,
        grid=(GRID,),
        in_specs=[
            pl.BlockSpec((RB, d_in), lambda i: (i, 0)),
            pl.BlockSpec((2, RB, d_in), lambda i: (0, i, 0)),
            pl.BlockSpec((2, RB, 128), lambda i: (0, i, 0)),
        ],
        out_specs=pl.BlockSpec((RB, d_in), lambda i: (i, 0)),
        out_shape=jax.ShapeDtypeStruct((N_PAD, d_in), jnp.float32),
    )(s, a, d)


def _make_sc_agg(w, edge_split=False):
    mesh = plsc.VectorSubcoreMesh(
        core_axis_name="c", subcore_axis_name="s", num_cores=NC, num_subcores=NS)
    nchunks = CHUNKS // 2 if edge_split else CHUNKS

    out_type = [jax.ShapeDtypeStruct((2, N_PAD, w), jnp.float32)]
    scratch = [
        pltpu.VMEM((2, CK), jnp.int32),
        pltpu.VMEM((2, CK), jnp.int32),
        pltpu.VMEM((2, CK, w), jnp.float32),
        pltpu.VMEM_SHARED((N_PAD, w), jnp.float32),
        pltpu.SemaphoreType.DMA((2,)),
        pltpu.SemaphoreType.DMA((2,)),
        pltpu.SemaphoreType.DMA((2,)),
    ]

    def body(m_hbm, src_hbm, dst_hbm, z_hbm, agg_hbm, *rest):
        (srcb, dstb, rows_v, agg_sh, g_sem, s_sem, d_sem) = rest
        c = lax.axis_index("c")
        s = lax.axis_index("s")
        if edge_split:
            my_src = src_hbm.at[c * NS + s]
            my_dst = dst_hbm.at[c * NS + s]
        else:
            my_src = src_hbm.at[s]
            my_dst = dst_hbm.at[s]

        def zagg(k, _):
            pltpu.sync_copy(z_hbm, agg_sh.at[pl.ds(s * ROWS_PER_TILE + k * 64, 64)])
            return 0
        lax.fori_loop(0, ROWS_PER_TILE // 64, zagg, 0)

        plsc.subcore_barrier()

        m_view = m_hbm if edge_split else m_hbm.at[c]

        pltpu.async_copy(my_src.at[0], srcb.at[0], s_sem.at[0])
        pltpu.async_copy(my_dst.at[0], dstb.at[0], d_sem.at[0])
        pltpu.make_async_copy(my_src.at[0], srcb.at[0], s_sem.at[0]).wait()
        pltpu.async_copy(m_view.at[srcb.at[0]], rows_v.at[0], g_sem.at[0])
        pltpu.async_copy(my_src.at[1], srcb.at[1], s_sem.at[1])
        pltpu.async_copy(my_dst.at[1], dstb.at[1], d_sem.at[1])

        def pair(g, _):
            for b in (0, 1):
                j = g * 2 + b
                nb = 1 - b
                pltpu.make_async_copy(
                    m_view.at[srcb.at[b]], rows_v.at[b], g_sem.at[b]).wait()

                @pl.when(j < nchunks - 1)
                def _():
                    pltpu.make_async_copy(
                        my_src.at[j + 1], srcb.at[nb], s_sem.at[nb]).wait()
                    pltpu.async_copy(m_view.at[srcb.at[nb]], rows_v.at[nb],
                                     g_sem.at[nb])

                    @pl.when(j < nchunks - 2)
                    def _():
                        pltpu.async_copy(my_src.at[j + 2], srcb.at[b],
                                         s_sem.at[b])

                pltpu.make_async_copy(my_dst.at[j], dstb.at[b], d_sem.at[b]).wait()
                pltpu.sync_copy(rows_v.at[b], agg_sh.at[dstb.at[b]], add=True)

                @pl.when(j < nchunks - 2)
                def _():
                    pltpu.async_copy(my_dst.at[j + 2], dstb.at[b], d_sem.at[b])
            return 0
        lax.fori_loop(0, nchunks // 2, pair, 0)

        plsc.subcore_barrier()

        rsl = pl.ds(s * ROWS_PER_TILE, ROWS_PER_TILE)
        pltpu.sync_copy(agg_sh.at[rsl], agg_hbm.at[c].at[rsl])

    return pl.kernel(body, out_type=out_type, mesh=mesh, scratch_types=scratch)


def _make_sc_deg():
    mesh = plsc.VectorSubcoreMesh(
        core_axis_name="c", subcore_axis_name="s", num_cores=NC, num_subcores=NS)
    nchunks = CHUNKS // 2
    w = 128

    out_type = [jax.ShapeDtypeStruct((2, N_PAD, w), jnp.float32)]
    scratch = [
        pltpu.VMEM((2, CK), jnp.int32),
        pltpu.VMEM((CK, w), jnp.float32),
        pltpu.VMEM_SHARED((N_PAD, w), jnp.float32),
        pltpu.SemaphoreType.DMA((2,)),
    ]

    def body(dst_hbm, z_hbm, deg_hbm, dstb, ones_v, deg_sh, d_sem):
        c = lax.axis_index("c")
        s = lax.axis_index("s")
        my_dst = dst_hbm.at[c * NS + s]

        def onesrow(r, _):
            def onescol(i, _):
                ones_v[r, pl.ds(i * 16, 16)] = jnp.ones((16,), jnp.float32)
                return 0
            return lax.fori_loop(0, w // 16, onescol, 0)
        lax.fori_loop(0, CK, onesrow, 0)

        def zdg(k, _):
            pltpu.sync_copy(z_hbm, deg_sh.at[pl.ds(s * ROWS_PER_TILE + k * 64, 64)])
            return 0
        lax.fori_loop(0, ROWS_PER_TILE // 64, zdg, 0)

        plsc.subcore_barrier()

        pltpu.async_copy(my_dst.at[0], dstb.at[0], d_sem.at[0])
        pltpu.async_copy(my_dst.at[1], dstb.at[1], d_sem.at[1])

        def pair(g, _):
            for b in (0, 1):
                j = g * 2 + b
                pltpu.make_async_copy(my_dst.at[j], dstb.at[b], d_sem.at[b]).wait()
                pltpu.sync_copy(ones_v, deg_sh.at[dstb.at[b]], add=True)

                @pl.when(j < nchunks - 2)
                def _():
                    pltpu.async_copy(my_dst.at[j + 2], dstb.at[b], d_sem.at[b])
            return 0
        lax.fori_loop(0, nchunks // 2, pair, 0)

        plsc.subcore_barrier()

        rsl = pl.ds(s * ROWS_PER_TILE, ROWS_PER_TILE)
        pltpu.sync_copy(deg_sh.at[rsl], deg_hbm.at[c].at[rsl])

    return pl.kernel(body, out_type=out_type, mesh=mesh, scratch_types=scratch)


def kernel(x, edge_index, W_self0, W_neigh0, b0, W_self1, W_neigh1, b1,
           W_self2, W_neigh2, b2):
    f32 = jnp.float32
    xp = jnp.zeros((N_PAD, D_IN), f32).at[:N].set(x)
    src = jnp.concatenate(
        [edge_index[0], jnp.zeros((E_PAD - E,), jnp.int32)]).reshape(NS, CHUNKS, CK)
    dst = jnp.concatenate(
        [edge_index[1], jnp.full((E_PAD - E,), PAD_DST, jnp.int32)]).reshape(NS, CHUNKS, CK)
    ws2 = jnp.zeros((D_H, D_OUT_PAD), f32).at[:, :D_OUT].set(W_self2)
    wn2 = jnp.zeros((D_H, D_OUT_PAD), f32).at[:, :D_OUT].set(W_neigh2)
    b2p = jnp.zeros((1, D_OUT_PAD), f32).at[0, :D_OUT].set(b2)

    src2 = src.reshape(NC * NS, CHUNKS // 2, CK)
    dst2 = dst.reshape(NC * NS, CHUNKS // 2, CK)
    z128 = jnp.zeros((64, D_H // 2), f32)

    s0, m0 = _mm0_call(xp, W_self0, W_neigh0, b0.reshape(1, -1))
    [degp] = _make_sc_deg()(dst2, z128)
    agg0 = _make_sc_agg(D_H // 2)(m0, src, dst, z128)[0]
    s1, m1 = _comb_call(s0, agg0, degp, W_self1, W_neigh1, b1.reshape(1, -1), D_H)
    [agg1] = _make_sc_agg(D_H // 2)(m1, src, dst, z128)
    s2, m2 = _comb_flat_call(s1, agg1, degp, ws2, wn2, b2p, D_OUT_PAD)
    [agg2] = _make_sc_agg(D_OUT_PAD, edge_split=True)(m2, src2, dst2, z128)
    out = _fin_call(s2, agg2, degp)
    return out[:N, :D_OUT]


# speedup vs baseline: 2.7514x; 1.0000x over previous
import jax
import jax.numpy as jnp
from jax import lax
from jax.experimental import pallas as pl
from jax.experimental.pallas import tpu as pltpu
from jax.experimental.pallas import tpu_sc as plsc

N = 10000
E = 160000
D_IN = 256
D_H = 256
D_OUT = 40
D_OUT_PAD = 128

NC = 2
NS = 16
CK = 64
CHUNKS = 160
E_PAD = NS * CHUNKS * CK
N_PAD = 10240
ROWS_PER_TILE = N_PAD // NS
PAD_DST = 10016
RB = 512
GRID = N_PAD // RB


def _tc_mm0(x_ref, ws_ref, wn_ref, b_ref, s_ref, m_ref):
    xb = x_ref[...]
    s_ref[...] = jnp.dot(xb, ws_ref[...], preferred_element_type=jnp.float32) + b_ref[...]
    mm = jnp.dot(xb, wn_ref[...], preferred_element_type=jnp.float32)
    half = mm.shape[1] // 2
    m_ref[0] = mm[:, :half]
    m_ref[1] = mm[:, half:]


def _tc_comb(s_ref, a_ref, d_ref, ws_ref, wn_ref, b_ref, s_out, m_out):
    deg = jnp.maximum(d_ref[0, :, 0:1] + d_ref[1, :, 0:1], 1.0)
    h = s_ref[...] + jnp.concatenate([a_ref[0], a_ref[1]], axis=1) / deg
    h = jnp.maximum(h, 0.0)
    s_out[...] = jnp.dot(h, ws_ref[...], preferred_element_type=jnp.float32) + b_ref[...]
    mm = jnp.dot(h, wn_ref[...], preferred_element_type=jnp.float32)
    half = mm.shape[1] // 2
    m_out[0] = mm[:, :half]
    m_out[1] = mm[:, half:]


def _tc_comb_flat(s_ref, a_ref, d_ref, ws_ref, wn_ref, b_ref, s_out, m_out):
    deg = jnp.maximum(d_ref[0, :, 0:1] + d_ref[1, :, 0:1], 1.0)
    h = s_ref[...] + jnp.concatenate([a_ref[0], a_ref[1]], axis=1) / deg
    h = jnp.maximum(h, 0.0)
    s_out[...] = jnp.dot(h, ws_ref[...], preferred_element_type=jnp.float32) + b_ref[...]
    m_out[...] = jnp.dot(h, wn_ref[...], preferred_element_type=jnp.float32)


def _tc_fin(s_ref, a_ref, d_ref, o_ref):
    deg = jnp.maximum(d_ref[0, :, 0:1] + d_ref[1, :, 0:1], 1.0)
    o_ref[...] = s_ref[...] + (a_ref[0] + a_ref[1]) / deg


def _mm0_call(x, ws, wn, b):
    return pl.pallas_call(
        _tc_mm0,
        grid=(GRID,),
        in_specs=[
            pl.BlockSpec((RB, D_IN), lambda i: (i, 0)),
            pl.BlockSpec((D_IN, D_H), lambda i: (0, 0)),
            pl.BlockSpec((D_IN, D_H), lambda i: (0, 0)),
            pl.BlockSpec((1, D_H), lambda i: (0, 0)),
        ],
        out_specs=[
            pl.BlockSpec((RB, D_H), lambda i: (i, 0)),
            pl.BlockSpec((2, RB, D_H // 2), lambda i: (0, i, 0)),
        ],
        out_shape=[
            jax.ShapeDtypeStruct((N_PAD, D_H), jnp.float32),
            jax.ShapeDtypeStruct((2, N_PAD, D_H // 2), jnp.float32),
        ],
    )(x, ws, wn, b)


def _comb_call(s, a, d, ws, wn, b, d_out):
    d_in = s.shape[1]
    return pl.pallas_call(
        _tc_comb,
        grid=(GRID,),
        in_specs=[
            pl.BlockSpec((RB, d_in), lambda i: (i, 0)),
            pl.BlockSpec((2, RB, d_in // 2), lambda i: (0, i, 0)),
            pl.BlockSpec((2, RB, 128), lambda i: (0, i, 0)),
            pl.BlockSpec((d_in, d_out), lambda i: (0, 0)),
            pl.BlockSpec((d_in, d_out), lambda i: (0, 0)),
            pl.BlockSpec((1, d_out), lambda i: (0, 0)),
        ],
        out_specs=[
            pl.BlockSpec((RB, d_out), lambda i: (i, 0)),
            pl.BlockSpec((2, RB, d_out // 2), lambda i: (0, i, 0)),
        ],
        out_shape=[
            jax.ShapeDtypeStruct((N_PAD, d_out), jnp.float32),
            jax.ShapeDtypeStruct((2, N_PAD, d_out // 2), jnp.float32),
        ],
    )(s, a, d, ws, wn, b)


def _comb_flat_call(s, a, d, ws, wn, b, d_out):
    d_in = s.shape[1]
    return pl.pallas_call(
        _tc_comb_flat,
        grid=(GRID,),
        in_specs=[
            pl.BlockSpec((RB, d_in), lambda i: (i, 0)),
            pl.BlockSpec((2, RB, d_in // 2), lambda i: (0, i, 0)),
            pl.BlockSpec((2, RB, 128), lambda i: (0, i, 0)),
            pl.BlockSpec((d_in, d_out), lambda i: (0, 0)),
            pl.BlockSpec((d_in, d_out), lambda i: (0, 0)),
            pl.BlockSpec((1, d_out), lambda i: (0, 0)),
        ],
        out_specs=[
            pl.BlockSpec((RB, d_out), lambda i: (i, 0)),
            pl.BlockSpec((RB, d_out), lambda i: (i, 0)),
        ],
        out_shape=[
            jax.ShapeDtypeStruct((N_PAD, d_out), jnp.float32),
            jax.ShapeDtypeStruct((N_PAD, d_out), jnp.float32),
        ],
    )(s, a, d, ws, wn, b)


def _fin_call(s, a, d):
    d_in = s.shape[1]
    return pl.pallas_call(
        _tc_fin,
        grid=(GRID,),
        in_specs=[
            pl.BlockSpec((RB, d_in), lambda i: (i, 0)),
            pl.BlockSpec((2, RB, d_in), lambda i: (0, i, 0)),
            pl.BlockSpec((2, RB, 128), lambda i: (0, i, 0)),
        ],
        out_specs=pl.BlockSpec((RB, d_in), lambda i: (i, 0)),
        out_shape=jax.ShapeDtypeStruct((N_PAD, d_in), jnp.float32),
    )(s, a, d)


def _make_sc_agg(w, edge_split=False):
    mesh = plsc.VectorSubcoreMesh(
        core_axis_name="c", subcore_axis_name="s", num_cores=NC, num_subcores=NS)
    nchunks = CHUNKS // 2 if edge_split else CHUNKS

    out_type = [jax.ShapeDtypeStruct((2, N_PAD, w), jnp.float32)]
    scratch = [
        pltpu.VMEM((2, CK), jnp.int32),
        pltpu.VMEM((4, CK), jnp.int32),
        pltpu.VMEM((2, CK, w), jnp.float32),
        pltpu.VMEM_SHARED((N_PAD, w), jnp.float32),
        pltpu.SemaphoreType.DMA((2,)),
        pltpu.SemaphoreType.DMA((2,)),
        pltpu.SemaphoreType.DMA((2,)),
        pltpu.SemaphoreType.DMA((4,)),
    ]

    def body(m_hbm, src_hbm, dst_hbm, z_hbm, agg_hbm, *rest):
        (srcb, dstb, rows_v, agg_sh, g_sem, sc_sem, s_sem, d_sem) = rest
        c = lax.axis_index("c")
        s = lax.axis_index("s")
        if edge_split:
            my_src = src_hbm.at[c * NS + s]
            my_dst = dst_hbm.at[c * NS + s]
        else:
            my_src = src_hbm.at[s]
            my_dst = dst_hbm.at[s]

        pltpu.async_copy(my_src.at[0], srcb.at[0], s_sem.at[0])
        for d in (0, 1, 2):
            pltpu.async_copy(my_dst.at[d], dstb.at[d], d_sem.at[d])

        def zagg(k, _):
            pltpu.sync_copy(z_hbm, agg_sh.at[pl.ds(s * ROWS_PER_TILE + k * 64, 64)])
            return 0
        lax.fori_loop(0, ROWS_PER_TILE // 64, zagg, 0)

        plsc.subcore_barrier()

        m_view = m_hbm if edge_split else m_hbm.at[c]

        pltpu.make_async_copy(my_src.at[0], srcb.at[0], s_sem.at[0]).wait()
        pltpu.async_copy(m_view.at[srcb.at[0]], rows_v.at[0], g_sem.at[0])
        pltpu.async_copy(my_src.at[1], srcb.at[1], s_sem.at[1])

        def quad(g, _):
            for b in (0, 1, 2, 3):
                j = g * 4 + b
                r = b % 2
                nr = 1 - r
                nd = (b + 3) % 4
                pltpu.make_async_copy(
                    m_view.at[srcb.at[r]], rows_v.at[r], g_sem.at[r]).wait()
                pltpu.make_async_copy(my_dst.at[j], dstb.at[b], d_sem.at[b]).wait()
                pltpu.async_copy(rows_v.at[r], agg_sh.at[dstb.at[b]],
                                 sc_sem.at[r], add=True)

                @pl.when(j < nchunks - 1)
                def _():
                    @pl.when(j > 0)
                    def _():
                        pltpu.make_async_copy(
                            rows_v.at[nr], agg_sh.at[dstb.at[nd]],
                            sc_sem.at[nr]).wait()
                    pltpu.make_async_copy(
                        my_src.at[j + 1], srcb.at[nr], s_sem.at[nr]).wait()
                    pltpu.async_copy(m_view.at[srcb.at[nr]], rows_v.at[nr],
                                     g_sem.at[nr])

                    @pl.when(j < nchunks - 2)
                    def _():
                        pltpu.async_copy(my_src.at[j + 2], srcb.at[r],
                                         s_sem.at[r])

                    @pl.when(j < nchunks - 3)
                    def _():
                        pltpu.async_copy(my_dst.at[j + 3], dstb.at[nd],
                                         d_sem.at[nd])
            return 0
        lax.fori_loop(0, nchunks // 4, quad, 0)

        pltpu.make_async_copy(rows_v.at[0], agg_sh.at[dstb.at[2]],
                              sc_sem.at[0]).wait()
        pltpu.make_async_copy(rows_v.at[1], agg_sh.at[dstb.at[3]],
                              sc_sem.at[1]).wait()

        plsc.subcore_barrier()

        rsl = pl.ds(s * ROWS_PER_TILE, ROWS_PER_TILE)
        pltpu.sync_copy(agg_sh.at[rsl], agg_hbm.at[c].at[rsl])

    return pl.kernel(body, out_type=out_type, mesh=mesh, scratch_types=scratch)


def _make_sc_deg():
    mesh = plsc.VectorSubcoreMesh(
        core_axis_name="c", subcore_axis_name="s", num_cores=NC, num_subcores=NS)
    nchunks = CHUNKS // 2
    w = 128

    out_type = [jax.ShapeDtypeStruct((2, N_PAD, w), jnp.float32)]
    scratch = [
        pltpu.VMEM((2, CK), jnp.int32),
        pltpu.VMEM((CK, w), jnp.float32),
        pltpu.VMEM_SHARED((N_PAD, w), jnp.float32),
        pltpu.SemaphoreType.DMA((2,)),
    ]

    def body(dst_hbm, z_hbm, deg_hbm, dstb, ones_v, deg_sh, d_sem):
        c = lax.axis_index("c")
        s = lax.axis_index("s")
        my_dst = dst_hbm.at[c * NS + s]

        def onesrow(r, _):
            def onescol(i, _):
                ones_v[r, pl.ds(i * 16, 16)] = jnp.ones((16,), jnp.float32)
                return 0
            return lax.fori_loop(0, w // 16, onescol, 0)
        lax.fori_loop(0, CK, onesrow, 0)

        def zdg(k, _):
            pltpu.sync_copy(z_hbm, deg_sh.at[pl.ds(s * ROWS_PER_TILE + k * 64, 64)])
            return 0
        lax.fori_loop(0, ROWS_PER_TILE // 64, zdg, 0)

        plsc.subcore_barrier()

        pltpu.async_copy(my_dst.at[0], dstb.at[0], d_sem.at[0])
        pltpu.async_copy(my_dst.at[1], dstb.at[1], d_sem.at[1])

        def pair(g, _):
            for b in (0, 1):
                j = g * 2 + b
                pltpu.make_async_copy(my_dst.at[j], dstb.at[b], d_sem.at[b]).wait()
                pltpu.sync_copy(ones_v, deg_sh.at[dstb.at[b]], add=True)

                @pl.when(j < nchunks - 2)
                def _():
                    pltpu.async_copy(my_dst.at[j + 2], dstb.at[b], d_sem.at[b])
            return 0
        lax.fori_loop(0, nchunks // 2, pair, 0)

        plsc.subcore_barrier()

        rsl = pl.ds(s * ROWS_PER_TILE, ROWS_PER_TILE)
        pltpu.sync_copy(deg_sh.at[rsl], deg_hbm.at[c].at[rsl])

    return pl.kernel(body, out_type=out_type, mesh=mesh, scratch_types=scratch)


def kernel(x, edge_index, W_self0, W_neigh0, b0, W_self1, W_neigh1, b1,
           W_self2, W_neigh2, b2):
    f32 = jnp.float32
    xp = jnp.zeros((N_PAD, D_IN), f32).at[:N].set(x)
    src = jnp.concatenate(
        [edge_index[0], jnp.zeros((E_PAD - E,), jnp.int32)]).reshape(NS, CHUNKS, CK)
    dst = jnp.concatenate(
        [edge_index[1], jnp.full((E_PAD - E,), PAD_DST, jnp.int32)]).reshape(NS, CHUNKS, CK)
    ws2 = jnp.zeros((D_H, D_OUT_PAD), f32).at[:, :D_OUT].set(W_self2)
    wn2 = jnp.zeros((D_H, D_OUT_PAD), f32).at[:, :D_OUT].set(W_neigh2)
    b2p = jnp.zeros((1, D_OUT_PAD), f32).at[0, :D_OUT].set(b2)

    src2 = src.reshape(NC * NS, CHUNKS // 2, CK)
    dst2 = dst.reshape(NC * NS, CHUNKS // 2, CK)
    z128 = jnp.zeros((64, D_H // 2), f32)

    s0, m0 = _mm0_call(xp, W_self0, W_neigh0, b0.reshape(1, -1))
    [degp] = _make_sc_deg()(dst2, z128)
    agg0 = _make_sc_agg(D_H // 2)(m0, src, dst, z128)[0]
    s1, m1 = _comb_call(s0, agg0, degp, W_self1, W_neigh1, b1.reshape(1, -1), D_H)
    [agg1] = _make_sc_agg(D_H // 2)(m1, src, dst, z128)
    s2, m2 = _comb_flat_call(s1, agg1, degp, ws2, wn2, b2p, D_OUT_PAD)
    [agg2] = _make_sc_agg(D_OUT_PAD, edge_split=True)(m2, src2, dst2, z128)
    out = _fin_call(s2, agg2, degp)
    return out[:N, :D_OUT]


# speedup vs baseline: 3.1052x; 1.1286x over previous
import jax
import jax.numpy as jnp
from jax import lax
from jax.experimental import pallas as pl
from jax.experimental.pallas import tpu as pltpu
from jax.experimental.pallas import tpu_sc as plsc

N = 10000
E = 160000
D_IN = 256
D_H = 256
D_OUT = 40
D_OUT_PAD = 128

NC = 2
NS = 16
CK = 64
CHUNKS = 160
E_PAD = NS * CHUNKS * CK
N_PAD = 10240
ROWS_PER_TILE = N_PAD // NS
PAD_DST = 10016
RB = 512
GRID = N_PAD // RB


def _tc_mm0(x_ref, ws_ref, wn_ref, b_ref, s_ref, m_ref):
    xb = x_ref[...]
    s_ref[...] = jnp.dot(xb, ws_ref[...], preferred_element_type=jnp.float32) + b_ref[...]
    mm = jnp.dot(xb, wn_ref[...], preferred_element_type=jnp.float32)
    half = mm.shape[1] // 2
    m_ref[0] = mm[:, :half]
    m_ref[1] = mm[:, half:]


def _tc_comb(s_ref, a_ref, d_ref, ws_ref, wn_ref, b_ref, s_out, m_out):
    deg = jnp.maximum(d_ref[0, :, 0:1] + d_ref[1, :, 0:1], 1.0)
    h = s_ref[...] + jnp.concatenate([a_ref[0], a_ref[1]], axis=1) / deg
    h = jnp.maximum(h, 0.0)
    s_out[...] = jnp.dot(h, ws_ref[...], preferred_element_type=jnp.float32) + b_ref[...]
    mm = jnp.dot(h, wn_ref[...], preferred_element_type=jnp.float32)
    half = mm.shape[1] // 2
    m_out[0] = mm[:, :half]
    m_out[1] = mm[:, half:]


def _tc_comb_flat(s_ref, a_ref, d_ref, ws_ref, wn_ref, b_ref, s_out, m_out):
    deg = jnp.maximum(d_ref[0, :, 0:1] + d_ref[1, :, 0:1], 1.0)
    h = s_ref[...] + jnp.concatenate([a_ref[0], a_ref[1]], axis=1) / deg
    h = jnp.maximum(h, 0.0)
    s_out[...] = jnp.dot(h, ws_ref[...], preferred_element_type=jnp.float32) + b_ref[...]
    m_out[...] = jnp.dot(h, wn_ref[...], preferred_element_type=jnp.float32)


def _tc_fin(s_ref, a_ref, d_ref, o_ref):
    deg = jnp.maximum(d_ref[0, :, 0:1] + d_ref[1, :, 0:1], 1.0)
    o_ref[...] = s_ref[...] + (a_ref[0] + a_ref[1]) / deg


def _mm0_call(x, ws, wn, b):
    return pl.pallas_call(
        _tc_mm0,
        grid=(GRID,),
        in_specs=[
            pl.BlockSpec((RB, D_IN), lambda i: (i, 0)),
            pl.BlockSpec((D_IN, D_H), lambda i: (0, 0)),
            pl.BlockSpec((D_IN, D_H), lambda i: (0, 0)),
            pl.BlockSpec((1, D_H), lambda i: (0, 0)),
        ],
        out_specs=[
            pl.BlockSpec((RB, D_H), lambda i: (i, 0)),
            pl.BlockSpec((2, RB, D_H // 2), lambda i: (0, i, 0)),
        ],
        out_shape=[
            jax.ShapeDtypeStruct((N_PAD, D_H), jnp.float32),
            jax.ShapeDtypeStruct((2, N_PAD, D_H // 2), jnp.float32),
        ],
    )(x, ws, wn, b)


def _comb_call(s, a, d, ws, wn, b, d_out):
    d_in = s.shape[1]
    return pl.pallas_call(
        _tc_comb,
        grid=(GRID,),
        in_specs=[
            pl.BlockSpec((RB, d_in), lambda i: (i, 0)),
            pl.BlockSpec((2, RB, d_in // 2), lambda i: (0, i, 0)),
            pl.BlockSpec((2, RB, 128), lambda i: (0, i, 0)),
            pl.BlockSpec((d_in, d_out), lambda i: (0, 0)),
            pl.BlockSpec((d_in, d_out), lambda i: (0, 0)),
            pl.BlockSpec((1, d_out), lambda i: (0, 0)),
        ],
        out_specs=[
            pl.BlockSpec((RB, d_out), lambda i: (i, 0)),
            pl.BlockSpec((2, RB, d_out // 2), lambda i: (0, i, 0)),
        ],
        out_shape=[
            jax.ShapeDtypeStruct((N_PAD, d_out), jnp.float32),
            jax.ShapeDtypeStruct((2, N_PAD, d_out // 2), jnp.float32),
        ],
    )(s, a, d, ws, wn, b)


def _comb_flat_call(s, a, d, ws, wn, b, d_out):
    d_in = s.shape[1]
    return pl.pallas_call(
        _tc_comb_flat,
        grid=(GRID,),
        in_specs=[
            pl.BlockSpec((RB, d_in), lambda i: (i, 0)),
            pl.BlockSpec((2, RB, d_in // 2), lambda i: (0, i, 0)),
            pl.BlockSpec((2, RB, 128), lambda i: (0, i, 0)),
            pl.BlockSpec((d_in, d_out), lambda i: (0, 0)),
            pl.BlockSpec((d_in, d_out), lambda i: (0, 0)),
            pl.BlockSpec((1, d_out), lambda i: (0, 0)),
        ],
        out_specs=[
            pl.BlockSpec((RB, d_out), lambda i: (i, 0)),
            pl.BlockSpec((RB, d_out), lambda i: (i, 0)),
        ],
        out_shape=[
            jax.ShapeDtypeStruct((N_PAD, d_out), jnp.float32),
            jax.ShapeDtypeStruct((N_PAD, d_out), jnp.float32),
        ],
    )(s, a, d, ws, wn, b)


def _fin_call(s, a, d):
    d_in = s.shape[1]
    return pl.pallas_call(
        _tc_fin,
        grid=(GRID,),
        in_specs=[
            pl.BlockSpec((RB, d_in), lambda i: (i, 0)),
            pl.BlockSpec((2, RB, d_in), lambda i: (0, i, 0)),
            pl.BlockSpec((2, RB, 128), lambda i: (0, i, 0)),
        ],
        out_specs=pl.BlockSpec((RB, d_in), lambda i: (i, 0)),
        out_shape=jax.ShapeDtypeStruct((N_PAD, d_in), jnp.float32),
    )(s, a, d)


def _make_sc_agg(w, edge_split=False):
    mesh = plsc.VectorSubcoreMesh(
        core_axis_name="c", subcore_axis_name="s", num_cores=NC, num_subcores=NS)
    nchunks = CHUNKS // 2 if edge_split else CHUNKS

    out_type = [jax.ShapeDtypeStruct((2, N_PAD, w), jnp.float32)]
    scratch = [
        pltpu.VMEM((8, CK), jnp.int32),
        pltpu.VMEM((8, CK), jnp.int32),
        pltpu.VMEM((4, CK, w), jnp.float32),
        pltpu.VMEM_SHARED((N_PAD, w), jnp.float32),
        pltpu.SemaphoreType.DMA((4,)),
        pltpu.SemaphoreType.DMA((4,)),
        pltpu.SemaphoreType.DMA((8,)),
        pltpu.SemaphoreType.DMA((8,)),
    ]

    def body(m_hbm, src_hbm, dst_hbm, z_hbm, agg_hbm, *rest):
        (srcb, dstb, rows_v, agg_sh, g_sem, sc_sem, s_sem, d_sem) = rest
        c = lax.axis_index("c")
        s = lax.axis_index("s")
        if edge_split:
            my_src = src_hbm.at[c * NS + s]
            my_dst = dst_hbm.at[c * NS + s]
        else:
            my_src = src_hbm.at[s]
            my_dst = dst_hbm.at[s]

        for d in range(6):
            pltpu.async_copy(my_src.at[d], srcb.at[d], s_sem.at[d])
            pltpu.async_copy(my_dst.at[d], dstb.at[d], d_sem.at[d])

        def zagg(k, _):
            pltpu.sync_copy(z_hbm, agg_sh.at[pl.ds(s * ROWS_PER_TILE + k * 64, 64)])
            return 0
        lax.fori_loop(0, ROWS_PER_TILE // 64, zagg, 0)

        plsc.subcore_barrier()

        m_view = m_hbm if edge_split else m_hbm.at[c]

        for q in (0, 1, 2):
            pltpu.make_async_copy(my_src.at[q], srcb.at[q], s_sem.at[q]).wait()
            pltpu.async_copy(m_view.at[srcb.at[q]], rows_v.at[q], g_sem.at[q])

        def oct_(g, _):
            for b in range(8):
                j = g * 8 + b
                r4 = b % 4
                pltpu.make_async_copy(
                    m_view.at[srcb.at[b]], rows_v.at[r4], g_sem.at[r4]).wait()
                pltpu.make_async_copy(my_dst.at[j], dstb.at[b], d_sem.at[b]).wait()
                pltpu.async_copy(rows_v.at[r4], agg_sh.at[dstb.at[b]],
                                 sc_sem.at[r4], add=True)

                @pl.when(j + 3 < nchunks)
                def _():
                    @pl.when(j > 0)
                    def _():
                        pltpu.make_async_copy(
                            rows_v.at[(r4 + 3) % 4],
                            agg_sh.at[dstb.at[(b + 7) % 8]],
                            sc_sem.at[(r4 + 3) % 4]).wait()
                    pltpu.make_async_copy(
                        my_src.at[j + 3], srcb.at[(b + 3) % 8],
                        s_sem.at[(b + 3) % 8]).wait()
                    pltpu.async_copy(m_view.at[srcb.at[(b + 3) % 8]],
                                     rows_v.at[(r4 + 3) % 4],
                                     g_sem.at[(r4 + 3) % 4])

                    @pl.when(j + 6 < nchunks)
                    def _():
                        pltpu.async_copy(my_src.at[j + 6], srcb.at[(b + 6) % 8],
                                         s_sem.at[(b + 6) % 8])
                        pltpu.async_copy(my_dst.at[j + 6], dstb.at[(b + 6) % 8],
                                         d_sem.at[(b + 6) % 8])
            return 0
        lax.fori_loop(0, nchunks // 8, oct_, 0)

        pltpu.make_async_copy(rows_v.at[0], agg_sh.at[dstb.at[4]],
                              sc_sem.at[0]).wait()
        pltpu.make_async_copy(rows_v.at[1], agg_sh.at[dstb.at[5]],
                              sc_sem.at[1]).wait()
        pltpu.make_async_copy(rows_v.at[2], agg_sh.at[dstb.at[6]],
                              sc_sem.at[2]).wait()
        pltpu.make_async_copy(rows_v.at[3], agg_sh.at[dstb.at[7]],
                              sc_sem.at[3]).wait()

        plsc.subcore_barrier()

        rsl = pl.ds(s * ROWS_PER_TILE, ROWS_PER_TILE)
        pltpu.sync_copy(agg_sh.at[rsl], agg_hbm.at[c].at[rsl])

    return pl.kernel(body, out_type=out_type, mesh=mesh, scratch_types=scratch)


def _make_sc_deg():
    mesh = plsc.VectorSubcoreMesh(
        core_axis_name="c", subcore_axis_name="s", num_cores=NC, num_subcores=NS)
    nchunks = CHUNKS // 2
    w = 128

    out_type = [jax.ShapeDtypeStruct((2, N_PAD, w), jnp.float32)]
    scratch = [
        pltpu.VMEM((2, CK), jnp.int32),
        pltpu.VMEM((CK, w), jnp.float32),
        pltpu.VMEM_SHARED((N_PAD, w), jnp.float32),
        pltpu.SemaphoreType.DMA((2,)),
    ]

    def body(dst_hbm, z_hbm, deg_hbm, dstb, ones_v, deg_sh, d_sem):
        c = lax.axis_index("c")
        s = lax.axis_index("s")
        my_dst = dst_hbm.at[c * NS + s]

        def onesrow(r, _):
            def onescol(i, _):
                ones_v[r, pl.ds(i * 16, 16)] = jnp.ones((16,), jnp.float32)
                return 0
            return lax.fori_loop(0, w // 16, onescol, 0)
        lax.fori_loop(0, CK, onesrow, 0)

        def zdg(k, _):
            pltpu.sync_copy(z_hbm, deg_sh.at[pl.ds(s * ROWS_PER_TILE + k * 64, 64)])
            return 0
        lax.fori_loop(0, ROWS_PER_TILE // 64, zdg, 0)

        plsc.subcore_barrier()

        pltpu.async_copy(my_dst.at[0], dstb.at[0], d_sem.at[0])
        pltpu.async_copy(my_dst.at[1], dstb.at[1], d_sem.at[1])

        def pair(g, _):
            for b in (0, 1):
                j = g * 2 + b
                pltpu.make_async_copy(my_dst.at[j], dstb.at[b], d_sem.at[b]).wait()
                pltpu.sync_copy(ones_v, deg_sh.at[dstb.at[b]], add=True)

                @pl.when(j < nchunks - 2)
                def _():
                    pltpu.async_copy(my_dst.at[j + 2], dstb.at[b], d_sem.at[b])
            return 0
        lax.fori_loop(0, nchunks // 2, pair, 0)

        plsc.subcore_barrier()

        rsl = pl.ds(s * ROWS_PER_TILE, ROWS_PER_TILE)
        pltpu.sync_copy(deg_sh.at[rsl], deg_hbm.at[c].at[rsl])

    return pl.kernel(body, out_type=out_type, mesh=mesh, scratch_types=scratch)


def kernel(x, edge_index, W_self0, W_neigh0, b0, W_self1, W_neigh1, b1,
           W_self2, W_neigh2, b2):
    f32 = jnp.float32
    xp = jnp.zeros((N_PAD, D_IN), f32).at[:N].set(x)
    src = jnp.concatenate(
        [edge_index[0], jnp.zeros((E_PAD - E,), jnp.int32)]).reshape(NS, CHUNKS, CK)
    dst = jnp.concatenate(
        [edge_index[1], jnp.full((E_PAD - E,), PAD_DST, jnp.int32)]).reshape(NS, CHUNKS, CK)
    ws2 = jnp.zeros((D_H, D_OUT_PAD), f32).at[:, :D_OUT].set(W_self2)
    wn2 = jnp.zeros((D_H, D_OUT_PAD), f32).at[:, :D_OUT].set(W_neigh2)
    b2p = jnp.zeros((1, D_OUT_PAD), f32).at[0, :D_OUT].set(b2)

    src2 = src.reshape(NC * NS, CHUNKS // 2, CK)
    dst2 = dst.reshape(NC * NS, CHUNKS // 2, CK)
    z128 = jnp.zeros((64, D_H // 2), f32)

    s0, m0 = _mm0_call(xp, W_self0, W_neigh0, b0.reshape(1, -1))
    [degp] = _make_sc_deg()(dst2, z128)
    agg0 = _make_sc_agg(D_H // 2)(m0, src, dst, z128)[0]
    s1, m1 = _comb_call(s0, agg0, degp, W_self1, W_neigh1, b1.reshape(1, -1), D_H)
    [agg1] = _make_sc_agg(D_H // 2)(m1, src, dst, z128)
    s2, m2 = _comb_flat_call(s1, agg1, degp, ws2, wn2, b2p, D_OUT_PAD)
    [agg2] = _make_sc_agg(D_OUT_PAD, edge_split=True)(m2, src2, dst2, z128)
    out = _fin_call(s2, agg2, degp)
    return out[:N, :D_OUT]
